# Initial kernel scaffold; baseline (speedup 1.0000x reference)
import jax
import jax.numpy as jnp
from jax.experimental import pallas as pl


def kernel(x, edge_index, W1, b1, gamma, beta, W2, b2, Wfc, bfc):
    raise NotImplementedError("write your pallas kernel here")


# speedup vs baseline: 5.4238x; 5.4238x over previous
import functools

import jax
import jax.numpy as jnp
from jax import lax
from jax.experimental import pallas as pl
from jax.experimental.pallas import tpu as pltpu
from jax.experimental.pallas import tpu_sc as plsc

N_NODES = 10000
N_EDGES = 320000
NFEAT = 128
NCLASS = 64
BN_EPS = 1e-5

_NC = 2
_NS = 16
_EDGES_PER_TILE = N_EDGES // (_NC * _NS)
_CHUNK = 80
_N_CHUNKS = _EDGES_PER_TILE // _CHUNK
_ROWS_MAIN = 624
_ROWS_LAST = N_NODES - 15 * _ROWS_MAIN


def _sc_segment_sum(x, src, dst):
    mesh = plsc.VectorSubcoreMesh(core_axis_name="c", subcore_axis_name="s")

    @functools.partial(
        pl.kernel,
        mesh=mesh,
        out_type=jax.ShapeDtypeStruct((_NC * N_NODES, NFEAT), jnp.float32),
        scratch_types=[
            pltpu.VMEM((_CHUNK,), jnp.int32),
            pltpu.VMEM((_CHUNK,), jnp.int32),
            pltpu.VMEM((_CHUNK, NFEAT), jnp.float32),
            pltpu.VMEM_SHARED((N_NODES, NFEAT), jnp.float32),
            pltpu.SemaphoreType.DMA,
        ],
    )
    def k(x_hbm, src_hbm, dst_hbm, out_hbm, src_v, dst_v, rows_v, agg_sh, sem):
        c = lax.axis_index("c")
        s = lax.axis_index("s")
        row0 = s * _ROWS_MAIN
        @pl.when(s < _NS - 1)
        def _():
            pltpu.sync_copy(x_hbm.at[pl.ds(row0, _ROWS_MAIN)],
                            agg_sh.at[pl.ds(row0, _ROWS_MAIN)])

        @pl.when(s == _NS - 1)
        def _():
            pltpu.sync_copy(x_hbm.at[pl.ds(row0, _ROWS_LAST)],
                            agg_sh.at[pl.ds(row0, _ROWS_LAST)])

        plsc.subcore_barrier()

        wid = c * _NS + s
        ebase = wid * _EDGES_PER_TILE

        def body(i, carry):
            base = ebase + i * _CHUNK
            pltpu.sync_copy(src_hbm.at[pl.ds(base, _CHUNK)], src_v)
            pltpu.sync_copy(dst_hbm.at[pl.ds(base, _CHUNK)], dst_v)
            pltpu.async_copy(x_hbm.at[src_v], rows_v, sem).wait()
            pltpu.sync_copy(rows_v, agg_sh.at[dst_v], add=True)
            return carry

        lax.fori_loop(0, _N_CHUNKS, body, 0)
        plsc.subcore_barrier()

        @pl.when(s < _NS - 1)
        def _():
            pltpu.sync_copy(agg_sh.at[pl.ds(row0, _ROWS_MAIN)],
                            out_hbm.at[pl.ds(c * N_NODES + row0, _ROWS_MAIN)])

        @pl.when(s == _NS - 1)
        def _():
            pltpu.sync_copy(agg_sh.at[pl.ds(row0, _ROWS_LAST)],
                            out_hbm.at[pl.ds(c * N_NODES + row0, _ROWS_LAST)])

    return k(x, src, dst)


def _tc_mlp(p, x, W1, b1, gamma, beta, W2, b2, Wfc, bfc):
    def body(p_ref, x_ref, w1_ref, b1_ref, g_ref, be_ref, w2_ref, b2_ref,
             wfc_ref, bfc_ref, o_ref):
        h = p_ref[0:N_NODES, :] + p_ref[N_NODES:2 * N_NODES, :] - x_ref[...]
        h = jnp.dot(h, w1_ref[...], preferred_element_type=jnp.float32) + b1_ref[...]
        h = jnp.maximum(h, 0.0)
        mean = jnp.mean(h, axis=0, keepdims=True)
        d = h - mean
        var = jnp.mean(d * d, axis=0, keepdims=True)
        h = d * (g_ref[...] * jax.lax.rsqrt(var + BN_EPS)) + be_ref[...]
        h = jnp.dot(h, w2_ref[...], preferred_element_type=jnp.float32) + b2_ref[...]
        o_ref[...] = (jnp.dot(h, wfc_ref[...], preferred_element_type=jnp.float32)
                      + bfc_ref[...])

    return pl.pallas_call(
        body,
        out_shape=jax.ShapeDtypeStruct((N_NODES, NCLASS), jnp.float32),
    )(p, x, W1, b1.reshape(1, -1), gamma.reshape(1, -1), beta.reshape(1, -1),
      W2, b2.reshape(1, -1), Wfc, bfc.reshape(1, -1))


def kernel(x, edge_index, W1, b1, gamma, beta, W2, b2, Wfc, bfc):
    src = edge_index[0]
    dst = edge_index[1]
    p = _sc_segment_sum(x, src, dst)
    return _tc_mlp(p, x, W1, b1, gamma, beta, W2, b2, Wfc, bfc)


# speedup vs baseline: 10.5781x; 1.9503x over previous
import functools

import jax
import jax.numpy as jnp
from jax import lax
from jax.experimental import pallas as pl
from jax.experimental.pallas import tpu as pltpu
from jax.experimental.pallas import tpu_sc as plsc

N_NODES = 10000
N_EDGES = 320000
NFEAT = 128
NCLASS = 64
BN_EPS = 1e-5

_NC = 2
_NS = 16
_EDGES_PER_TILE = N_EDGES // (_NC * _NS)
_T = 125
_TPS = 8
_SUPER = _T * _TPS
_NSUPER = _EDGES_PER_TILE // _SUPER
_ROWS_MAIN = 624
_ROWS_LAST = N_NODES - 15 * _ROWS_MAIN


def _sc_segment_sum(x, src, dst):
    mesh = plsc.VectorSubcoreMesh(core_axis_name="c", subcore_axis_name="s")

    @functools.partial(
        pl.kernel,
        mesh=mesh,
        out_type=jax.ShapeDtypeStruct((_NC * N_NODES, NFEAT), jnp.float32),
        scratch_types=[
            pltpu.VMEM((1, _TPS, _T), jnp.int32),
            pltpu.VMEM((1, _TPS, _T), jnp.int32),
            pltpu.VMEM((_T, NFEAT), jnp.float32),
            pltpu.VMEM((_T, NFEAT), jnp.float32),
            pltpu.VMEM_SHARED((N_NODES, NFEAT), jnp.float32),
            pltpu.SemaphoreType.DMA,
            pltpu.SemaphoreType.DMA,
            pltpu.SemaphoreType.DMA,
            pltpu.SemaphoreType.DMA,
        ],
    )
    def k(x_hbm, src_hbm, dst_hbm, out_hbm, sidx, didx, rows_a, rows_b,
          agg_sh, gsem_a, gsem_b, ssem_a, ssem_b):
        c = lax.axis_index("c")
        s = lax.axis_index("s")
        row0 = s * _ROWS_MAIN
        @pl.when(s < _NS - 1)
        def _():
            pltpu.sync_copy(x_hbm.at[pl.ds(row0, _ROWS_MAIN)],
                            agg_sh.at[pl.ds(row0, _ROWS_MAIN)])

        @pl.when(s == _NS - 1)
        def _():
            pltpu.sync_copy(x_hbm.at[pl.ds(row0, _ROWS_LAST)],
                            agg_sh.at[pl.ds(row0, _ROWS_LAST)])

        plsc.subcore_barrier()

        wid = c * _NS + s
        qbase = wid * _NSUPER
        rows = (rows_a, rows_b)
        gsem = (gsem_a, gsem_b)
        ssem = (ssem_a, ssem_b)

        def body(j, carry):
            q = qbase + j
            pltpu.sync_copy(src_hbm.at[pl.ds(q, 1)], sidx)
            pltpu.sync_copy(dst_hbm.at[pl.ds(q, 1)], didx)
            g = [None] * _TPS
            sc = [None] * _TPS
            g[0] = pltpu.async_copy(x_hbm.at[sidx.at[0, 0]], rows[0], gsem[0])
            for t in range(1, _TPS):
                b = t % 2
                if t >= 2:
                    sc[t - 2].wait()
                g[t] = pltpu.async_copy(x_hbm.at[sidx.at[0, t]], rows[b], gsem[b])
                g[t - 1].wait()
                sc[t - 1] = pltpu.async_copy(
                    rows[1 - b], agg_sh.at[didx.at[0, t - 1]], ssem[1 - b],
                    add=True)
            g[_TPS - 1].wait()
            sc[_TPS - 1] = pltpu.async_copy(
                rows[(_TPS - 1) % 2], agg_sh.at[didx.at[0, _TPS - 1]],
                ssem[(_TPS - 1) % 2], add=True)
            sc[_TPS - 2].wait()
            sc[_TPS - 1].wait()
            return carry

        lax.fori_loop(0, _NSUPER, body, 0)
        plsc.subcore_barrier()

        @pl.when(s < _NS - 1)
        def _():
            pltpu.sync_copy(agg_sh.at[pl.ds(row0, _ROWS_MAIN)],
                            out_hbm.at[pl.ds(c * N_NODES + row0, _ROWS_MAIN)])

        @pl.when(s == _NS - 1)
        def _():
            pltpu.sync_copy(agg_sh.at[pl.ds(row0, _ROWS_LAST)],
                            out_hbm.at[pl.ds(c * N_NODES + row0, _ROWS_LAST)])

    return k(x, src, dst)


def _tc_mlp(p, x, W1, b1, gamma, beta, W2, b2, Wfc, bfc):
    def body(p_ref, x_ref, w1_ref, b1_ref, g_ref, be_ref, w2_ref, b2_ref,
             wfc_ref, bfc_ref, o_ref):
        h = p_ref[0:N_NODES, :] + p_ref[N_NODES:2 * N_NODES, :] - x_ref[...]
        h = jnp.dot(h, w1_ref[...], preferred_element_type=jnp.float32) + b1_ref[...]
        h = jnp.maximum(h, 0.0)
        mean = jnp.mean(h, axis=0, keepdims=True)
        d = h - mean
        var = jnp.mean(d * d, axis=0, keepdims=True)
        h = d * (g_ref[...] * jax.lax.rsqrt(var + BN_EPS)) + be_ref[...]
        h = jnp.dot(h, w2_ref[...], preferred_element_type=jnp.float32) + b2_ref[...]
        o_ref[...] = (jnp.dot(h, wfc_ref[...], preferred_element_type=jnp.float32)
                      + bfc_ref[...])

    return pl.pallas_call(
        body,
        out_shape=jax.ShapeDtypeStruct((N_NODES, NCLASS), jnp.float32),
    )(p, x, W1, b1.reshape(1, -1), gamma.reshape(1, -1), beta.reshape(1, -1),
      W2, b2.reshape(1, -1), Wfc, bfc.reshape(1, -1))


def kernel(x, edge_index, W1, b1, gamma, beta, W2, b2, Wfc, bfc):
    src = edge_index[0].reshape(N_EDGES // (_TPS * _T), _TPS, _T)
    dst = edge_index[1].reshape(N_EDGES // (_TPS * _T), _TPS, _T)
    p = _sc_segment_sum(x, src, dst)
    return _tc_mlp(p, x, W1, b1, gamma, beta, W2, b2, Wfc, bfc)


# speedup vs baseline: 11.9047x; 1.1254x over previous
import functools

import jax
import jax.numpy as jnp
from jax import lax
from jax.experimental import pallas as pl
from jax.experimental.pallas import tpu as pltpu
from jax.experimental.pallas import tpu_sc as plsc

N_NODES = 10000
N_EDGES = 320000
NFEAT = 128
NCLASS = 64
BN_EPS = 1e-5

_NC = 2
_NS = 16
_EDGES_PER_TILE = N_EDGES // (_NC * _NS)
_T = 100
_NT = _EDGES_PER_TILE // _T
_TPB = 10
_NBODY = _NT // _TPB
_NBUF = 2
_HALF = _NT // 2
_ROWS_MAIN = 624
_ROWS_LAST = N_NODES - 15 * _ROWS_MAIN


def _sc_segment_sum(x, e4):
    mesh = plsc.VectorSubcoreMesh(core_axis_name="c", subcore_axis_name="s")

    @functools.partial(
        pl.kernel,
        mesh=mesh,
        out_type=jax.ShapeDtypeStruct((_NC * N_NODES, NFEAT), jnp.float32),
        scratch_types=[
            pltpu.VMEM((_HALF, _T), jnp.int32),
            pltpu.VMEM((_HALF, _T), jnp.int32),
            pltpu.VMEM((_NBUF, _T, NFEAT), jnp.float32),
            pltpu.VMEM_SHARED((N_NODES, NFEAT), jnp.float32),
            pltpu.SemaphoreType.DMA,
            pltpu.SemaphoreType.DMA,
            pltpu.SemaphoreType.DMA,
            pltpu.SemaphoreType.DMA,
        ],
    )
    def k(x_hbm, e_hbm, out_hbm, sidx, didx, rows, agg_sh,
          gsem_0, gsem_1, ssem_0, ssem_1):
        c = lax.axis_index("c")
        s = lax.axis_index("s")
        wid = c * _NS + s
        row0 = s * _ROWS_MAIN
        pltpu.sync_copy(e_hbm.at[0, wid, 0], sidx)
        pltpu.sync_copy(e_hbm.at[1, wid, 0], didx)

        @pl.when(s < _NS - 1)
        def _():
            pltpu.sync_copy(x_hbm.at[pl.ds(row0, _ROWS_MAIN)],
                            agg_sh.at[pl.ds(row0, _ROWS_MAIN)])

        @pl.when(s == _NS - 1)
        def _():
            pltpu.sync_copy(x_hbm.at[pl.ds(row0, _ROWS_LAST)],
                            agg_sh.at[pl.ds(row0, _ROWS_LAST)])

        plsc.subcore_barrier()

        gsem = (gsem_0, gsem_1)
        ssem = (ssem_0, ssem_1)

        def body(j, carry):
            @pl.when(j == _NBODY // 2)
            def _():
                pltpu.sync_copy(e_hbm.at[0, wid, 1], sidx)
                pltpu.sync_copy(e_hbm.at[1, wid, 1], didx)

            t0 = j * _TPB - (j // (_NBODY // 2)) * _HALF
            g = [None] * _TPB
            sc = [None] * _TPB
            for t in range(_TPB):
                b = t % _NBUF
                if t >= _NBUF:
                    sc[t - _NBUF].wait()
                g[t] = pltpu.async_copy(x_hbm.at[sidx.at[t0 + t]],
                                        rows.at[b], gsem[b])
                if t >= 1:
                    g[t - 1].wait()
                    sc[t - 1] = pltpu.async_copy(
                        rows.at[(t - 1) % _NBUF],
                        agg_sh.at[didx.at[t0 + t - 1]],
                        ssem[(t - 1) % _NBUF], add=True)
            g[_TPB - 1].wait()
            sc[_TPB - 1] = pltpu.async_copy(
                rows.at[(_TPB - 1) % _NBUF],
                agg_sh.at[didx.at[t0 + _TPB - 1]],
                ssem[(_TPB - 1) % _NBUF], add=True)
            for t in range(_TPB - _NBUF, _TPB):
                sc[t].wait()
            return carry

        lax.fori_loop(0, _NBODY, body, 0)
        plsc.subcore_barrier()

        @pl.when(s < _NS - 1)
        def _():
            pltpu.sync_copy(agg_sh.at[pl.ds(row0, _ROWS_MAIN)],
                            out_hbm.at[pl.ds(c * N_NODES + row0, _ROWS_MAIN)])

        @pl.when(s == _NS - 1)
        def _():
            pltpu.sync_copy(agg_sh.at[pl.ds(row0, _ROWS_LAST)],
                            out_hbm.at[pl.ds(c * N_NODES + row0, _ROWS_LAST)])

    return k(x, e4)


def _tc_mlp(p, x, W1, b1, gamma, beta, W2, b2, Wfc, bfc):
    def body(p_ref, x_ref, w1_ref, b1_ref, g_ref, be_ref, w2_ref, b2_ref,
             wfc_ref, bfc_ref, o_ref):
        h = p_ref[0:N_NODES, :] + p_ref[N_NODES:2 * N_NODES, :] - x_ref[...]
        h = jnp.dot(h, w1_ref[...], preferred_element_type=jnp.float32) + b1_ref[...]
        h = jnp.maximum(h, 0.0)
        mean = jnp.mean(h, axis=0, keepdims=True)
        d = h - mean
        var = jnp.mean(d * d, axis=0, keepdims=True)
        h = d * (g_ref[...] * jax.lax.rsqrt(var + BN_EPS)) + be_ref[...]
        h = jnp.dot(h, w2_ref[...], preferred_element_type=jnp.float32) + b2_ref[...]
        o_ref[...] = (jnp.dot(h, wfc_ref[...], preferred_element_type=jnp.float32)
                      + bfc_ref[...])

    return pl.pallas_call(
        body,
        out_shape=jax.ShapeDtypeStruct((N_NODES, NCLASS), jnp.float32),
    )(p, x, W1, b1.reshape(1, -1), gamma.reshape(1, -1), beta.reshape(1, -1),
      W2, b2.reshape(1, -1), Wfc, bfc.reshape(1, -1))


def kernel(x, edge_index, W1, b1, gamma, beta, W2, b2, Wfc, bfc):
    e4 = edge_index.reshape(2, _NC * _NS, 2, _HALF, _T)
    p = _sc_segment_sum(x, e4)
    return _tc_mlp(p, x, W1, b1, gamma, beta, W2, b2, Wfc, bfc)


# speedup vs baseline: 12.8740x; 1.0814x over previous
import functools

import jax
import jax.numpy as jnp
from jax import lax
from jax.experimental import pallas as pl
from jax.experimental.pallas import tpu as pltpu
from jax.experimental.pallas import tpu_sc as plsc

N_NODES = 10000
N_EDGES = 320000
NFEAT = 128
NCLASS = 64
BN_EPS = 1e-5

_NC = 2
_NS = 16
_EDGES_PER_TILE = N_EDGES // (_NC * _NS)
_T = 125
_NT = _EDGES_PER_TILE // _T
_HALFT = _NT // 2
_TPB = 4
_NBODYH = (_HALFT - _TPB) // _TPB
_NBUF = 2
_ROWS_MAIN = 624
_ROWS_LAST = N_NODES - 15 * _ROWS_MAIN


def _sc_segment_sum(x, e4):
    mesh = plsc.VectorSubcoreMesh(core_axis_name="c", subcore_axis_name="s")

    @functools.partial(
        pl.kernel,
        mesh=mesh,
        out_type=jax.ShapeDtypeStruct((_NC * N_NODES, NFEAT), jnp.float32),
        scratch_types=[
            pltpu.VMEM((_HALFT, _T), jnp.int32),
            pltpu.VMEM((_HALFT, _T), jnp.int32),
            pltpu.VMEM((_NBUF, _T, NFEAT), jnp.float32),
            pltpu.VMEM_SHARED((N_NODES, NFEAT), jnp.float32),
            pltpu.SemaphoreType.DMA,
            pltpu.SemaphoreType.DMA,
            pltpu.SemaphoreType.DMA,
            pltpu.SemaphoreType.DMA,
        ],
    )
    def k(x_hbm, e_hbm, out_hbm, sidx, didx, rows, agg_sh,
          gsem_0, gsem_1, ssem_0, ssem_1):
        c = lax.axis_index("c")
        s = lax.axis_index("s")
        wid = c * _NS + s
        row0 = s * _ROWS_MAIN
        gsem = (gsem_0, gsem_1)
        ssem = (ssem_0, ssem_1)

        def issue_gather(k, b):
            pltpu.async_copy(x_hbm.at[sidx.at[k]], rows.at[b], gsem[b])

        def wait_gather(b):
            pltpu.make_async_copy(x_hbm.at[sidx.at[0]], rows.at[b],
                                  gsem[b]).wait()

        def issue_scatter(k, b):
            pltpu.async_copy(rows.at[b], agg_sh.at[didx.at[k]], ssem[b],
                             add=True)

        def wait_scatter(b):
            pltpu.make_async_copy(rows.at[b], agg_sh.at[didx.at[0]],
                                  ssem[b]).wait()

        def body(j, carry):
            k0 = j * _TPB
            for t in range(_TPB):
                k = k0 + t
                b = t % _NBUF
                wait_gather(b)
                issue_scatter(k, b)
                wait_scatter(b)
                issue_gather(k + _NBUF, b)
            return carry

        def do_half():
            issue_gather(0, 0)
            issue_gather(1, 1)
            lax.fori_loop(0, _NBODYH, body, 0)
            for kk in range(_NBODYH * _TPB, _HALFT):
                b = kk % _NBUF
                wait_gather(b)
                issue_scatter(kk, b)
                if kk + _NBUF < _HALFT:
                    wait_scatter(b)
                    issue_gather(kk + _NBUF, b)
            for kk in range(_HALFT - _NBUF, _HALFT):
                wait_scatter(kk % _NBUF)

        ih_s = pltpu.async_copy(e_hbm.at[0, wid, 0], sidx, gsem_0)
        ih_d = pltpu.async_copy(e_hbm.at[1, wid, 0], didx, gsem_1)

        @pl.when(s < _NS - 1)
        def _():
            pltpu.sync_copy(x_hbm.at[pl.ds(row0, _ROWS_MAIN)],
                            agg_sh.at[pl.ds(row0, _ROWS_MAIN)])

        @pl.when(s == _NS - 1)
        def _():
            pltpu.sync_copy(x_hbm.at[pl.ds(row0, _ROWS_LAST)],
                            agg_sh.at[pl.ds(row0, _ROWS_LAST)])

        ih_s.wait()
        ih_d.wait()
        plsc.subcore_barrier()
        do_half()
        pltpu.sync_copy(e_hbm.at[0, wid, 1], sidx)
        pltpu.sync_copy(e_hbm.at[1, wid, 1], didx)
        do_half()
        plsc.subcore_barrier()

        @pl.when(s < _NS - 1)
        def _():
            pltpu.sync_copy(agg_sh.at[pl.ds(row0, _ROWS_MAIN)],
                            out_hbm.at[pl.ds(c * N_NODES + row0, _ROWS_MAIN)])

        @pl.when(s == _NS - 1)
        def _():
            pltpu.sync_copy(agg_sh.at[pl.ds(row0, _ROWS_LAST)],
                            out_hbm.at[pl.ds(c * N_NODES + row0, _ROWS_LAST)])

    return k(x, e4)


def _tc_mlp(p, x, W1, b1, gamma, beta, W2, b2, Wfc, bfc):
    def body(p_ref, x_ref, w1_ref, b1_ref, g_ref, be_ref, w2_ref, b2_ref,
             wfc_ref, bfc_ref, o_ref):
        h = p_ref[0:N_NODES, :] + p_ref[N_NODES:2 * N_NODES, :] - x_ref[...]
        h = jnp.dot(h, w1_ref[...], preferred_element_type=jnp.float32) + b1_ref[...]
        h = jnp.maximum(h, 0.0)
        mean = jnp.mean(h, axis=0, keepdims=True)
        d = h - mean
        var = jnp.mean(d * d, axis=0, keepdims=True)
        h = d * (g_ref[...] * jax.lax.rsqrt(var + BN_EPS)) + be_ref[...]
        h = jnp.dot(h, w2_ref[...], preferred_element_type=jnp.float32) + b2_ref[...]
        o_ref[...] = (jnp.dot(h, wfc_ref[...], preferred_element_type=jnp.float32)
                      + bfc_ref[...])

    return pl.pallas_call(
        body,
        out_shape=jax.ShapeDtypeStruct((N_NODES, NCLASS), jnp.float32),
    )(p, x, W1, b1.reshape(1, -1), gamma.reshape(1, -1), beta.reshape(1, -1),
      W2, b2.reshape(1, -1), Wfc, bfc.reshape(1, -1))


def kernel(x, edge_index, W1, b1, gamma, beta, W2, b2, Wfc, bfc):
    e4 = edge_index.reshape(2, _NC * _NS, 2, _HALFT, _T)
    p = _sc_segment_sum(x, e4)
    return _tc_mlp(p, x, W1, b1, gamma, beta, W2, b2, Wfc, bfc)
